# baseline (speedup 1.0000x reference)
import functools
import jax
import jax.numpy as jnp
from jax import lax
from jax.experimental import pallas as pl
from jax.experimental.pallas import tpu as pltpu
from jax.experimental.pallas import tpu_sc as plsc

_B, _T, _C, _H, _M = 2, 2048, 1024, 16, 8192
_DH = _C // _H
_K = 3

_NC, _NS = 2, 16
_NW = _NC * _NS


def _qkv_body(x_ref, wq_ref, wk_ref, wv_ref, q_ref, kvm_ref, qb_ref, kvb_ref):
    xt = x_ref[...]
    q = jnp.dot(xt, wq_ref[...], preferred_element_type=jnp.float32)
    k = jnp.dot(xt, wk_ref[...], preferred_element_type=jnp.float32)
    v = jnp.dot(xt, wv_ref[...], preferred_element_type=jnp.float32)
    q_ref[...] = q
    kvm_ref[:, 0, :] = k
    kvm_ref[:, 1, :] = v
    qb_ref[...] = q.astype(jnp.bfloat16)
    kvb_ref[:, 0, :] = k.astype(jnp.bfloat16)
    kvb_ref[:, 1, :] = v.astype(jnp.bfloat16)


def _qkv(x2d, wq, wk, wv):
    n = x2d.shape[0]
    tt = 512
    return pl.pallas_call(
        _qkv_body,
        grid=(n // tt,),
        in_specs=[
            pl.BlockSpec((tt, _C), lambda i: (i, 0)),
            pl.BlockSpec((_C, _C), lambda i: (0, 0)),
            pl.BlockSpec((_C, _C), lambda i: (0, 0)),
            pl.BlockSpec((_C, _C), lambda i: (0, 0)),
        ],
        out_specs=[
            pl.BlockSpec((tt, _C), lambda i: (i, 0)),
            pl.BlockSpec((tt, 2, _C), lambda i: (i, 0, 0)),
            pl.BlockSpec((tt, _C), lambda i: (i, 0)),
            pl.BlockSpec((tt, 2, _C), lambda i: (i, 0, 0)),
        ],
        out_shape=[
            jax.ShapeDtypeStruct((n, _C), jnp.float32),
            jax.ShapeDtypeStruct((n, 2, _C), jnp.float32),
            jax.ShapeDtypeStruct((n, _C), jnp.bfloat16),
            jax.ShapeDtypeStruct((n, 2, _C), jnp.bfloat16),
        ],
    )(x2d, wq, wk, wv)


_TT = 256
_MT = 2048


def _topk_body(q_ref, mk_ref, idx_ref, v1, v2, v3, i1, i2, i3):
    b = pl.program_id(0)
    mi = pl.program_id(1)
    ti = pl.program_id(2)

    @pl.when(mi == 0)
    def _():
        ninf = jnp.full((_TT, 1), -jnp.inf, jnp.float32)
        v1[ti] = ninf
        v2[ti] = ninf
        v3[ti] = ninf
        zero = jnp.zeros((_TT, 1), jnp.int32)
        i1[ti] = zero
        i2[ti] = zero
        i3[ti] = zero

    qt = q_ref[0]
    mk = mk_ref[0, :, 0, :]
    s = lax.dot_general(qt, mk, (((1,), (1,)), ((), ())),
                        preferred_element_type=jnp.float32)

    cv1, cv2, cv3 = v1[ti], v2[ti], v3[ti]
    ci1, ci2, ci3 = i1[ti], i2[ti], i3[ti]
    col = lax.broadcasted_iota(jnp.int32, (_TT, _MT), 1)
    base = mi * _MT + b * _M
    for _ in range(_K):
        mval = jnp.max(s, axis=1, keepdims=True)
        mloc = jnp.min(jnp.where(s == mval, col, _M),
                       axis=1, keepdims=True)
        s = jnp.where(col == mloc, -jnp.inf, s)
        gidx = mloc + base
        gt1 = mval > cv1
        gt2 = mval > cv2
        gt3 = mval > cv3
        g12 = gt1 | gt2
        nv1 = jnp.where(gt1, mval, cv1)
        ni1 = jnp.where(gt1, gidx, ci1)
        nv2 = jnp.where(gt1, cv1, jnp.where(gt2, mval, cv2))
        ni2 = jnp.where(gt1, ci1, jnp.where(gt2, gidx, ci2))
        nv3 = jnp.where(g12, cv2, jnp.where(gt3, mval, cv3))
        ni3 = jnp.where(g12, ci2, jnp.where(gt3, gidx, ci3))
        cv1, cv2, cv3 = nv1, nv2, nv3
        ci1, ci2, ci3 = ni1, ni2, ni3

    v1[ti], v2[ti], v3[ti] = cv1, cv2, cv3
    i1[ti], i2[ti], i3[ti] = ci1, ci2, ci3
    idx_ref[0] = jnp.concatenate([ci1, ci2, ci3], axis=1)


def _sims_topk(q3d, mem_kv):
    nt = _T // _TT
    scr_f = pltpu.VMEM((nt, _TT, 1), jnp.float32)
    scr_i = pltpu.VMEM((nt, _TT, 1), jnp.int32)
    return pl.pallas_call(
        _topk_body,
        grid=(_B, _M // _MT, nt),
        in_specs=[
            pl.BlockSpec((1, _TT, _C), lambda b, mi, ti: (b, ti, 0)),
            pl.BlockSpec((1, _MT, 2, _C), lambda b, mi, ti: (b, mi, 0, 0)),
        ],
        out_specs=pl.BlockSpec((1, _TT, _K), lambda b, mi, ti: (b, ti, 0)),
        out_shape=jax.ShapeDtypeStruct((_B, _T, _K), jnp.int32),
        scratch_shapes=[scr_f, scr_f, scr_f, scr_i, scr_i, scr_i],
    )(q3d, mem_kv)


def _gather_rows(table, idx_flat):
    nidx = idx_flat.shape[0]
    b_per_w = nidx // _NW
    ch = 32
    d = table.shape[1]
    mesh = plsc.VectorSubcoreMesh(core_axis_name="c", subcore_axis_name="s")

    @functools.partial(
        pl.kernel,
        mesh=mesh,
        out_type=jax.ShapeDtypeStruct((nidx, d), jnp.float32),
        scratch_types=[
            pltpu.VMEM((b_per_w,), jnp.int32),
            pltpu.VMEM((ch, d), jnp.float32),
            pltpu.SemaphoreType.DMA,
        ],
    )
    def k(table_hbm, idx_hbm, out_hbm, idx_v, rows_v, sem):
        wid = lax.axis_index("s") * _NC + lax.axis_index("c")
        base = wid * b_per_w
        pltpu.sync_copy(idx_hbm.at[pl.ds(base, b_per_w)], idx_v)

        @pl.loop(0, b_per_w, step=ch)
        def _(o):
            pltpu.async_copy(table_hbm.at[idx_v.at[pl.ds(o, ch)]],
                             rows_v, sem).wait()
            pltpu.sync_copy(rows_v, out_hbm.at[pl.ds(base + o, ch)])

    return k(table, idx_flat)


_TQ = 128


def _sdpa_body(q_ref, kvm_ref, y_ref):
    qt = pl.program_id(1)
    row = qt * _TQ + lax.broadcasted_iota(jnp.int32, (_TQ, _T), 0)
    col = lax.broadcasted_iota(jnp.int32, (_TQ, _T), 1)
    mask = col <= row
    for h in range(_H):
        sl = slice(h * _DH, (h + 1) * _DH)
        qh = q_ref[0, :, sl]
        kh = kvm_ref[0, :, 0, sl]
        vh = kvm_ref[0, :, 1, sl]
        s = lax.dot_general(qh, kh, (((1,), (1,)), ((), ())),
                            preferred_element_type=jnp.float32) * 0.125
        s = jnp.where(mask, s, jnp.float32(-1e30))
        m = jnp.max(s, axis=1, keepdims=True)
        e = jnp.exp(s - m)
        den = jnp.sum(e, axis=1, keepdims=True)
        yh = jnp.dot(e.astype(jnp.bfloat16), vh,
                     preferred_element_type=jnp.float32)
        y_ref[0, :, sl] = yh / den


def _sdpa(q3d, kvm3d):
    return pl.pallas_call(
        _sdpa_body,
        grid=(_B, _T // _TQ),
        in_specs=[
            pl.BlockSpec((1, _TQ, _C), lambda b, i: (b, i, 0)),
            pl.BlockSpec((1, _T, 2, _C), lambda b, i: (b, 0, 0, 0)),
        ],
        out_specs=pl.BlockSpec((1, _TQ, _C), lambda b, i: (b, i, 0)),
        out_shape=jax.ShapeDtypeStruct((_B, _T, _C), jnp.float32),
    )(q3d, kvm3d)


_TF = 256


def _final_body(q_ref, g0_ref, g1_ref, g2_ref, y_ref, gate_ref, wp_ref,
                out_ref):
    g_refs = (g0_ref, g1_ref, g2_ref)
    lane_h = lax.broadcasted_iota(jnp.int32, (_C, _H), 0) // _DH
    head = lax.broadcasted_iota(jnp.int32, (_C, _H), 1)
    seg = (lane_h == head).astype(jnp.float32)
    lane_h_t = lax.broadcasted_iota(jnp.int32, (_H, _C), 1) // _DH
    head_t = lax.broadcasted_iota(jnp.int32, (_H, _C), 0)
    seg_t = (lane_h_t == head_t).astype(jnp.float32)

    q = q_ref[0]
    qk = []
    for k in range(_K):
        mk = g_refs[k][:, 0:_C]
        qk.append(jnp.dot(q * mk, seg,
                          preferred_element_type=jnp.float32) * 0.125)
    m = jnp.maximum(jnp.maximum(qk[0], qk[1]), qk[2])
    e = [jnp.exp(v - m) for v in qk]
    den = e[0] + e[1] + e[2]
    acc = jnp.zeros((_TF, _C), jnp.float32)
    for k in range(_K):
        w_full = jnp.dot(e[k] / den, seg_t,
                         preferred_element_type=jnp.float32)
        acc = acc + w_full * g_refs[k][:, _C:2 * _C]
    gate = jnp.dot(gate_ref[...], seg_t,
                   preferred_element_type=jnp.float32)
    combined = acc * gate + y_ref[0] * (1.0 - gate)
    out_ref[0] = jnp.dot(combined.astype(jnp.bfloat16),
                         wp_ref[...].astype(jnp.bfloat16),
                         preferred_element_type=jnp.float32)


def _final(q3d, gathered2d, y3d, gate_row, w_proj):
    nt = _T // _TF
    gspec = [
        pl.BlockSpec((_TF, 2 * _C),
                     functools.partial(lambda k, b, i: ((b * _K + k) * nt + i, 0), k))
        for k in range(_K)
    ]
    return pl.pallas_call(
        _final_body,
        grid=(_B, nt),
        in_specs=[
            pl.BlockSpec((1, _TF, _C), lambda b, i: (b, i, 0)),
            gspec[0], gspec[1], gspec[2],
            pl.BlockSpec((1, _TF, _C), lambda b, i: (b, i, 0)),
            pl.BlockSpec((1, _H), lambda b, i: (0, 0)),
            pl.BlockSpec((_C, _C), lambda b, i: (0, 0)),
        ],
        out_specs=pl.BlockSpec((1, _TF, _C), lambda b, i: (b, i, 0)),
        out_shape=jax.ShapeDtypeStruct((_B, _T, _C), jnp.float32),
    )(q3d, gathered2d, gathered2d, gathered2d, y3d, gate_row, w_proj)


@jax.jit
def kernel(x, mem_kv, W_attn, W_proj, gate_bias):
    b, t, c = x.shape
    wq = W_attn[:, :c]
    wk = W_attn[:, c:2 * c]
    wv = W_attn[:, 2 * c:]
    q2d, kvm2d, qb2d, kvb2d = _qkv(x.reshape(b * t, c), wq, wk, wv)
    q3d = q2d.reshape(b, t, c)
    kvm = kvm2d.reshape(b, t, 2, c)
    qb = qb2d.reshape(b, t, c)
    kvb = kvb2d.reshape(b, t, 2, c)

    idx = _sims_topk(q3d, mem_kv)
    idx_kmajor = idx.transpose(0, 2, 1).reshape(-1)
    gathered2d = _gather_rows(mem_kv.reshape(_B * _M, 2 * _C), idx_kmajor)
    y = _sdpa(qb, kvb)
    out = _final(q3d, gathered2d, y, gate_bias.reshape(1, _H), W_proj)
    return out, kvm


# speedup vs baseline: 2.1697x; 2.1697x over previous
import functools
import jax
import jax.numpy as jnp
from jax import lax
from jax.experimental import pallas as pl
from jax.experimental.pallas import tpu as pltpu
from jax.experimental.pallas import tpu_sc as plsc

_B, _T, _C, _H, _M = 2, 2048, 1024, 16, 8192
_DH = _C // _H
_K = 3

_NC, _NS = 2, 16
_NW = _NC * _NS


def _qkv_body(x_ref, wq_ref, wk_ref, wv_ref, q_ref, kvm_ref):
    xt = x_ref[...]
    q = jnp.dot(xt, wq_ref[...], preferred_element_type=jnp.float32)
    k = jnp.dot(xt, wk_ref[...], preferred_element_type=jnp.float32)
    v = jnp.dot(xt, wv_ref[...], preferred_element_type=jnp.float32)
    q_ref[...] = q
    kvm_ref[:, 0, :] = k
    kvm_ref[:, 1, :] = v


def _qkv(x2d, wq, wk, wv):
    n = x2d.shape[0]
    tt = 512
    return pl.pallas_call(
        _qkv_body,
        grid=(n // tt,),
        in_specs=[
            pl.BlockSpec((tt, _C), lambda i: (i, 0)),
            pl.BlockSpec((_C, _C), lambda i: (0, 0)),
            pl.BlockSpec((_C, _C), lambda i: (0, 0)),
            pl.BlockSpec((_C, _C), lambda i: (0, 0)),
        ],
        out_specs=[
            pl.BlockSpec((tt, _C), lambda i: (i, 0)),
            pl.BlockSpec((tt, 2, _C), lambda i: (i, 0, 0)),
        ],
        out_shape=[
            jax.ShapeDtypeStruct((n, _C), jnp.float32),
            jax.ShapeDtypeStruct((n, 2, _C), jnp.float32),
        ],
    )(x2d, wq, wk, wv)


_TT = 256
_MT = 2048


def _topk_body(q_ref, mk_ref, idx_ref, v1, v2, v3, i1, i2, i3):
    b = pl.program_id(0)
    mi = pl.program_id(1)
    ti = pl.program_id(2)

    @pl.when(mi == 0)
    def _():
        ninf = jnp.full((_TT, 1), -jnp.inf, jnp.float32)
        v1[ti] = ninf
        v2[ti] = ninf
        v3[ti] = ninf
        zero = jnp.zeros((_TT, 1), jnp.int32)
        i1[ti] = zero
        i2[ti] = zero
        i3[ti] = zero

    qt = q_ref[0]
    mk = mk_ref[0, :, 0, :]
    s = lax.dot_general(qt, mk, (((1,), (1,)), ((), ())),
                        preferred_element_type=jnp.float32)

    cv1, cv2, cv3 = v1[ti], v2[ti], v3[ti]
    ci1, ci2, ci3 = i1[ti], i2[ti], i3[ti]
    col = lax.broadcasted_iota(jnp.int32, (_TT, _MT), 1)
    base = mi * _MT + b * _M
    for _ in range(_K):
        mval = jnp.max(s, axis=1, keepdims=True)
        mloc = jnp.min(jnp.where(s == mval, col, _M),
                       axis=1, keepdims=True)
        s = jnp.where(col == mloc, -jnp.inf, s)
        gidx = mloc + base
        gt1 = mval > cv1
        gt2 = mval > cv2
        gt3 = mval > cv3
        g12 = gt1 | gt2
        nv1 = jnp.where(gt1, mval, cv1)
        ni1 = jnp.where(gt1, gidx, ci1)
        nv2 = jnp.where(gt1, cv1, jnp.where(gt2, mval, cv2))
        ni2 = jnp.where(gt1, ci1, jnp.where(gt2, gidx, ci2))
        nv3 = jnp.where(g12, cv2, jnp.where(gt3, mval, cv3))
        ni3 = jnp.where(g12, ci2, jnp.where(gt3, gidx, ci3))
        cv1, cv2, cv3 = nv1, nv2, nv3
        ci1, ci2, ci3 = ni1, ni2, ni3

    v1[ti], v2[ti], v3[ti] = cv1, cv2, cv3
    i1[ti], i2[ti], i3[ti] = ci1, ci2, ci3
    idx_ref[0] = jnp.concatenate([ci1, ci2, ci3], axis=1)


def _sims_topk(q3d, mem_kv):
    nt = _T // _TT
    scr_f = pltpu.VMEM((nt, _TT, 1), jnp.float32)
    scr_i = pltpu.VMEM((nt, _TT, 1), jnp.int32)
    return pl.pallas_call(
        _topk_body,
        grid=(_B, _M // _MT, nt),
        in_specs=[
            pl.BlockSpec((1, _TT, _C), lambda b, mi, ti: (b, ti, 0)),
            pl.BlockSpec((1, _MT, 2, _C), lambda b, mi, ti: (b, mi, 0, 0)),
        ],
        out_specs=pl.BlockSpec((1, _TT, _K), lambda b, mi, ti: (b, ti, 0)),
        out_shape=jax.ShapeDtypeStruct((_B, _T, _K), jnp.int32),
        scratch_shapes=[scr_f, scr_f, scr_f, scr_i, scr_i, scr_i],
    )(q3d, mem_kv)


def _gather_rows(table, idx_flat):
    nidx = idx_flat.shape[0]
    b_per_w = nidx // _NW
    ch = 32
    d = table.shape[1]
    mesh = plsc.VectorSubcoreMesh(core_axis_name="c", subcore_axis_name="s")

    @functools.partial(
        pl.kernel,
        mesh=mesh,
        out_type=jax.ShapeDtypeStruct((nidx, d), jnp.float32),
        scratch_types=[
            pltpu.VMEM((b_per_w,), jnp.int32),
            pltpu.VMEM((ch, d), jnp.float32),
            pltpu.SemaphoreType.DMA,
        ],
    )
    def k(table_hbm, idx_hbm, out_hbm, idx_v, rows_v, sem):
        wid = lax.axis_index("s") * _NC + lax.axis_index("c")
        base = wid * b_per_w
        pltpu.sync_copy(idx_hbm.at[pl.ds(base, b_per_w)], idx_v)

        @pl.loop(0, b_per_w, step=ch)
        def _(o):
            pltpu.async_copy(table_hbm.at[idx_v.at[pl.ds(o, ch)]],
                             rows_v, sem).wait()
            pltpu.sync_copy(rows_v, out_hbm.at[pl.ds(base + o, ch)])

    return k(table, idx_flat)


_TQ = 512


def _sdpa_body(q_ref, kvm_ref, y_ref):
    qt = pl.program_id(1)
    row = qt * _TQ + lax.broadcasted_iota(jnp.int32, (_TQ, _T), 0)
    col = lax.broadcasted_iota(jnp.int32, (_TQ, _T), 1)
    mask = col <= row
    for h in range(_H):
        sl = slice(h * _DH, (h + 1) * _DH)
        qh = q_ref[0, :, sl]
        kh = kvm_ref[0, :, 0, sl]
        vh = kvm_ref[0, :, 1, sl]
        s = lax.dot_general(qh, kh, (((1,), (1,)), ((), ())),
                            preferred_element_type=jnp.float32) * 0.125
        s = jnp.where(mask, s, jnp.float32(-1e30))
        m = jnp.max(s, axis=1, keepdims=True)
        e = jnp.exp(s - m)
        p = e / jnp.sum(e, axis=1, keepdims=True)
        y_ref[0, :, sl] = jnp.dot(p, vh, preferred_element_type=jnp.float32)


def _sdpa(q3d, kvm3d):
    return pl.pallas_call(
        _sdpa_body,
        grid=(_B, _T // _TQ),
        in_specs=[
            pl.BlockSpec((1, _TQ, _C), lambda b, i: (b, i, 0)),
            pl.BlockSpec((1, _T, 2, _C), lambda b, i: (b, 0, 0, 0)),
        ],
        out_specs=pl.BlockSpec((1, _TQ, _C), lambda b, i: (b, i, 0)),
        out_shape=jax.ShapeDtypeStruct((_B, _T, _C), jnp.float32),
    )(q3d, kvm3d)


_TF = 256


def _final_body(q_ref, g0_ref, g1_ref, g2_ref, y_ref, gate_ref, wp_ref,
                out_ref):
    g_refs = (g0_ref, g1_ref, g2_ref)
    lane_h = lax.broadcasted_iota(jnp.int32, (_C, _H), 0) // _DH
    head = lax.broadcasted_iota(jnp.int32, (_C, _H), 1)
    seg = (lane_h == head).astype(jnp.float32)
    lane_h_t = lax.broadcasted_iota(jnp.int32, (_H, _C), 1) // _DH
    head_t = lax.broadcasted_iota(jnp.int32, (_H, _C), 0)
    seg_t = (lane_h_t == head_t).astype(jnp.float32)

    q = q_ref[0]
    qk = []
    for k in range(_K):
        mk = g_refs[k][:, 0:_C]
        qk.append(jnp.dot(q * mk, seg,
                          preferred_element_type=jnp.float32) * 0.125)
    m = jnp.maximum(jnp.maximum(qk[0], qk[1]), qk[2])
    e = [jnp.exp(v - m) for v in qk]
    den = e[0] + e[1] + e[2]
    acc = jnp.zeros((_TF, _C), jnp.float32)
    for k in range(_K):
        w_full = jnp.dot(e[k] / den, seg_t,
                         preferred_element_type=jnp.float32)
        acc = acc + w_full * g_refs[k][:, _C:2 * _C]
    gate = jnp.dot(gate_ref[...], seg_t,
                   preferred_element_type=jnp.float32)
    combined = acc * gate + y_ref[0] * (1.0 - gate)
    out_ref[0] = jnp.dot(combined.astype(jnp.bfloat16),
                         wp_ref[...].astype(jnp.bfloat16),
                         preferred_element_type=jnp.float32)


def _final(q3d, gathered2d, y3d, gate_row, w_proj):
    nt = _T // _TF
    gspec = [
        pl.BlockSpec((_TF, 2 * _C),
                     functools.partial(lambda k, b, i: ((b * _K + k) * nt + i, 0), k))
        for k in range(_K)
    ]
    return pl.pallas_call(
        _final_body,
        grid=(_B, nt),
        in_specs=[
            pl.BlockSpec((1, _TF, _C), lambda b, i: (b, i, 0)),
            gspec[0], gspec[1], gspec[2],
            pl.BlockSpec((1, _TF, _C), lambda b, i: (b, i, 0)),
            pl.BlockSpec((1, _H), lambda b, i: (0, 0)),
            pl.BlockSpec((_C, _C), lambda b, i: (0, 0)),
        ],
        out_specs=pl.BlockSpec((1, _TF, _C), lambda b, i: (b, i, 0)),
        out_shape=jax.ShapeDtypeStruct((_B, _T, _C), jnp.float32),
    )(q3d, gathered2d, gathered2d, gathered2d, y3d, gate_row, w_proj)


@jax.jit
def kernel(x, mem_kv, W_attn, W_proj, gate_bias):
    b, t, c = x.shape
    wq = W_attn[:, :c]
    wk = W_attn[:, c:2 * c]
    wv = W_attn[:, 2 * c:]
    q2d, kvm2d = _qkv(x.reshape(b * t, c), wq, wk, wv)
    q3d = q2d.reshape(b, t, c)
    kvm = kvm2d.reshape(b, t, 2, c)

    idx = _sims_topk(q3d, mem_kv)
    idx_kmajor = idx.transpose(0, 2, 1).reshape(-1)
    gathered2d = _gather_rows(mem_kv.reshape(_B * _M, 2 * _C), idx_kmajor)
    y = _sdpa(q3d, kvm)
    out = _final(q3d, gathered2d, y, gate_bias.reshape(1, _H), W_proj)
    return out, kvm


# speedup vs baseline: 2.1974x; 1.0128x over previous
import functools
import jax
import jax.numpy as jnp
from jax import lax
from jax.experimental import pallas as pl
from jax.experimental.pallas import tpu as pltpu
from jax.experimental.pallas import tpu_sc as plsc

_B, _T, _C, _H, _M = 2, 2048, 1024, 16, 8192
_DH = _C // _H
_K = 3

_NC, _NS = 2, 16
_NW = _NC * _NS


def _qkv_body(x_ref, wq_ref, wk_ref, wv_ref, q_ref, kvm_ref,
              qb_ref, kb_ref, vb_ref):
    xt = x_ref[...]
    q = jnp.dot(xt, wq_ref[...], preferred_element_type=jnp.float32)
    k = jnp.dot(xt, wk_ref[...], preferred_element_type=jnp.float32)
    v = jnp.dot(xt, wv_ref[...], preferred_element_type=jnp.float32)
    q_ref[...] = q
    kvm_ref[:, 0, :] = k
    kvm_ref[:, 1, :] = v
    for h in range(_H):
        sl = slice(h * _DH, (h + 1) * _DH)
        qb_ref[h] = q[:, sl].astype(jnp.bfloat16)
        kb_ref[h] = k[:, sl].astype(jnp.bfloat16)
        vb_ref[h] = v[:, sl].astype(jnp.bfloat16)


def _qkv(x2d, wq, wk, wv):
    n = x2d.shape[0]
    tt = 512
    return pl.pallas_call(
        _qkv_body,
        grid=(n // tt,),
        in_specs=[
            pl.BlockSpec((tt, _C), lambda i: (i, 0)),
            pl.BlockSpec((_C, _C), lambda i: (0, 0)),
            pl.BlockSpec((_C, _C), lambda i: (0, 0)),
            pl.BlockSpec((_C, _C), lambda i: (0, 0)),
        ],
        out_specs=[
            pl.BlockSpec((tt, _C), lambda i: (i, 0)),
            pl.BlockSpec((tt, 2, _C), lambda i: (i, 0, 0)),
            pl.BlockSpec((_H, tt, _DH), lambda i: (0, i, 0)),
            pl.BlockSpec((_H, tt, _DH), lambda i: (0, i, 0)),
            pl.BlockSpec((_H, tt, _DH), lambda i: (0, i, 0)),
        ],
        out_shape=[
            jax.ShapeDtypeStruct((n, _C), jnp.float32),
            jax.ShapeDtypeStruct((n, 2, _C), jnp.float32),
            jax.ShapeDtypeStruct((_H, n, _DH), jnp.bfloat16),
            jax.ShapeDtypeStruct((_H, n, _DH), jnp.bfloat16),
            jax.ShapeDtypeStruct((_H, n, _DH), jnp.bfloat16),
        ],
    )(x2d, wq, wk, wv)


_TT = 256
_MT = 2048


def _topk_body(q_ref, mk_ref, idx_ref, v1, v2, v3, i1, i2, i3):
    b = pl.program_id(0)
    mi = pl.program_id(1)
    ti = pl.program_id(2)

    @pl.when(mi == 0)
    def _():
        ninf = jnp.full((_TT, 1), -jnp.inf, jnp.float32)
        v1[ti] = ninf
        v2[ti] = ninf
        v3[ti] = ninf
        zero = jnp.zeros((_TT, 1), jnp.int32)
        i1[ti] = zero
        i2[ti] = zero
        i3[ti] = zero

    qt = q_ref[0]
    mk = mk_ref[0, :, 0, :]
    s = lax.dot_general(qt, mk, (((1,), (1,)), ((), ())),
                        preferred_element_type=jnp.float32)

    cv1, cv2, cv3 = v1[ti], v2[ti], v3[ti]
    ci1, ci2, ci3 = i1[ti], i2[ti], i3[ti]
    col = lax.broadcasted_iota(jnp.int32, (_TT, _MT), 1)
    base = mi * _MT + b * _M
    for _ in range(_K):
        mval = jnp.max(s, axis=1, keepdims=True)
        mloc = jnp.min(jnp.where(s == mval, col, _M),
                       axis=1, keepdims=True)
        s = jnp.where(col == mloc, -jnp.inf, s)
        gidx = mloc + base
        gt1 = mval > cv1
        gt2 = mval > cv2
        gt3 = mval > cv3
        g12 = gt1 | gt2
        nv1 = jnp.where(gt1, mval, cv1)
        ni1 = jnp.where(gt1, gidx, ci1)
        nv2 = jnp.where(gt1, cv1, jnp.where(gt2, mval, cv2))
        ni2 = jnp.where(gt1, ci1, jnp.where(gt2, gidx, ci2))
        nv3 = jnp.where(g12, cv2, jnp.where(gt3, mval, cv3))
        ni3 = jnp.where(g12, ci2, jnp.where(gt3, gidx, ci3))
        cv1, cv2, cv3 = nv1, nv2, nv3
        ci1, ci2, ci3 = ni1, ni2, ni3

    v1[ti], v2[ti], v3[ti] = cv1, cv2, cv3
    i1[ti], i2[ti], i3[ti] = ci1, ci2, ci3
    idx_ref[0] = jnp.concatenate([ci1, ci2, ci3], axis=1)


def _sims_topk(q3d, mem_kv):
    nt = _T // _TT
    scr_f = pltpu.VMEM((nt, _TT, 1), jnp.float32)
    scr_i = pltpu.VMEM((nt, _TT, 1), jnp.int32)
    return pl.pallas_call(
        _topk_body,
        grid=(_B, _M // _MT, nt),
        in_specs=[
            pl.BlockSpec((1, _TT, _C), lambda b, mi, ti: (b, ti, 0)),
            pl.BlockSpec((1, _MT, 2, _C), lambda b, mi, ti: (b, mi, 0, 0)),
        ],
        out_specs=pl.BlockSpec((1, _TT, _K), lambda b, mi, ti: (b, ti, 0)),
        out_shape=jax.ShapeDtypeStruct((_B, _T, _K), jnp.int32),
        scratch_shapes=[scr_f, scr_f, scr_f, scr_i, scr_i, scr_i],
    )(q3d, mem_kv)


def _gather_rows(table, idx_flat):
    nidx = idx_flat.shape[0]
    b_per_w = nidx // _NW
    ch = 32
    d = table.shape[1]
    mesh = plsc.VectorSubcoreMesh(core_axis_name="c", subcore_axis_name="s")

    @functools.partial(
        pl.kernel,
        mesh=mesh,
        out_type=jax.ShapeDtypeStruct((nidx, d), jnp.float32),
        scratch_types=[
            pltpu.VMEM((b_per_w,), jnp.int32),
            pltpu.VMEM((ch, d), jnp.float32),
            pltpu.SemaphoreType.DMA,
        ],
    )
    def k(table_hbm, idx_hbm, out_hbm, idx_v, rows_v, sem):
        wid = lax.axis_index("s") * _NC + lax.axis_index("c")
        base = wid * b_per_w
        pltpu.sync_copy(idx_hbm.at[pl.ds(base, b_per_w)], idx_v)

        @pl.loop(0, b_per_w, step=ch)
        def _(o):
            pltpu.async_copy(table_hbm.at[idx_v.at[pl.ds(o, ch)]],
                             rows_v, sem).wait()
            pltpu.sync_copy(rows_v, out_hbm.at[pl.ds(base + o, ch)])

    return k(table, idx_flat)


_TQ = 512


def _sdpa_body(q_ref, k_ref, v_ref, y_ref):
    qt = pl.program_id(2)
    row = qt * _TQ + lax.broadcasted_iota(jnp.int32, (_TQ, _T), 0)
    col = lax.broadcasted_iota(jnp.int32, (_TQ, _T), 1)
    mask = col <= row
    qh = q_ref[0]
    kh = k_ref[0]
    vh = v_ref[0]
    s = lax.dot_general(qh, kh, (((1,), (1,)), ((), ())),
                        preferred_element_type=jnp.float32) * 0.125
    s = jnp.where(mask, s, jnp.float32(-1e30))
    m = jnp.max(s, axis=1, keepdims=True)
    e = jnp.exp(s - m)
    den = jnp.sum(e, axis=1, keepdims=True)
    yh = jnp.dot(e.astype(jnp.bfloat16), vh,
                 preferred_element_type=jnp.float32)
    y_ref[0] = yh / den


def _sdpa(qb, kb, vb):
    nq = _T // _TQ
    return pl.pallas_call(
        _sdpa_body,
        grid=(_B, _H, nq),
        in_specs=[
            pl.BlockSpec((1, _TQ, _DH), lambda b, h, i: (h, b * nq + i, 0)),
            pl.BlockSpec((1, _T, _DH), lambda b, h, i: (h, b, 0)),
            pl.BlockSpec((1, _T, _DH), lambda b, h, i: (h, b, 0)),
        ],
        out_specs=pl.BlockSpec((1, _TQ, _DH),
                               lambda b, h, i: (h, b * nq + i, 0)),
        out_shape=jax.ShapeDtypeStruct((_H, _B * _T, _DH), jnp.float32),
    )(qb, kb, vb)


_TF = 256


def _final_body(q_ref, g0_ref, g1_ref, g2_ref, yh_ref, gate_ref, wp_ref,
                out_ref):
    g_refs = (g0_ref, g1_ref, g2_ref)
    lane_h = lax.broadcasted_iota(jnp.int32, (_C, _H), 0) // _DH
    head = lax.broadcasted_iota(jnp.int32, (_C, _H), 1)
    seg = (lane_h == head).astype(jnp.float32)
    lane_h_t = lax.broadcasted_iota(jnp.int32, (_H, _C), 1) // _DH
    head_t = lax.broadcasted_iota(jnp.int32, (_H, _C), 0)
    seg_t = (lane_h_t == head_t).astype(jnp.float32)

    q = q_ref[0]
    qk = []
    for k in range(_K):
        mk = g_refs[k][:, 0:_C]
        qk.append(jnp.dot(q * mk, seg,
                          preferred_element_type=jnp.float32) * 0.125)
    m = jnp.maximum(jnp.maximum(qk[0], qk[1]), qk[2])
    e = [jnp.exp(v - m) for v in qk]
    den = e[0] + e[1] + e[2]
    acc = jnp.zeros((_TF, _C), jnp.float32)
    for k in range(_K):
        w_full = jnp.dot(e[k] / den, seg_t,
                         preferred_element_type=jnp.float32)
        acc = acc + w_full * g_refs[k][:, _C:2 * _C]
    gate = jnp.dot(gate_ref[...], seg_t,
                   preferred_element_type=jnp.float32)
    wp_b = wp_ref[...].astype(jnp.bfloat16)
    out = jnp.dot((acc * gate).astype(jnp.bfloat16), wp_b,
                  preferred_element_type=jnp.float32)
    for h in range(_H):
        sl = slice(h * _DH, (h + 1) * _DH)
        gh = gate_ref[0, h]
        yh = (yh_ref[h] * (1.0 - gh)).astype(jnp.bfloat16)
        out = out + jnp.dot(yh, wp_b[sl, :],
                            preferred_element_type=jnp.float32)
    out_ref[0] = out


def _final(q3d, gathered2d, yh, gate_row, w_proj):
    nt = _T // _TF
    gspec = [
        pl.BlockSpec((_TF, 2 * _C),
                     functools.partial(lambda k, b, i: ((b * _K + k) * nt + i, 0), k))
        for k in range(_K)
    ]
    return pl.pallas_call(
        _final_body,
        grid=(_B, nt),
        in_specs=[
            pl.BlockSpec((1, _TF, _C), lambda b, i: (b, i, 0)),
            gspec[0], gspec[1], gspec[2],
            pl.BlockSpec((_H, _TF, _DH), lambda b, i: (0, b * nt + i, 0)),
            pl.BlockSpec((1, _H), lambda b, i: (0, 0)),
            pl.BlockSpec((_C, _C), lambda b, i: (0, 0)),
        ],
        out_specs=pl.BlockSpec((1, _TF, _C), lambda b, i: (b, i, 0)),
        out_shape=jax.ShapeDtypeStruct((_B, _T, _C), jnp.float32),
    )(q3d, gathered2d, gathered2d, gathered2d, yh, gate_row, w_proj)


@jax.jit
def kernel(x, mem_kv, W_attn, W_proj, gate_bias):
    b, t, c = x.shape
    wq = W_attn[:, :c]
    wk = W_attn[:, c:2 * c]
    wv = W_attn[:, 2 * c:]
    q2d, kvm2d, qbh, kbh, vbh = _qkv(x.reshape(b * t, c), wq, wk, wv)
    q3d = q2d.reshape(b, t, c)
    kvm = kvm2d.reshape(b, t, 2, c)

    idx = _sims_topk(q3d, mem_kv)
    idx_kmajor = idx.transpose(0, 2, 1).reshape(-1)
    gathered2d = _gather_rows(mem_kv.reshape(_B * _M, 2 * _C), idx_kmajor)
    yh = _sdpa(qbh, kbh, vbh)
    out = _final(q3d, gathered2d, yh, gate_bias.reshape(1, _H), W_proj)
    return out, kvm
